# baseline (speedup 1.0000x reference)
import functools
import math

import jax
import jax.numpy as jnp
from jax import lax
from jax.experimental import pallas as pl
from jax.experimental.pallas import tpu as pltpu
from jax.experimental.pallas import tpu_sc as plsc

VOCAB = 100000
D_MODEL = 128
BATCH = 4096
HIST = 50
HIST_PAD = 56
PAIR = 2 * HIST_PAD

NC = 2
NS = 16
NW = NC * NS
B_PER_W = BATCH // NW
NPAIR = B_PER_W // 2
NRB = 8
SCALE = math.sqrt(D_MODEL)

_mesh = plsc.VectorSubcoreMesh(core_axis_name="c", subcore_axis_name="s")


@functools.partial(
    pl.kernel,
    mesh=_mesh,
    out_type=jax.ShapeDtypeStruct((BATCH // 2, PAIR, D_MODEL), jnp.float32),
    scratch_types=[
        pltpu.VMEM((NPAIR, PAIR), jnp.int32),
        pltpu.VMEM((NRB, PAIR, D_MODEL), jnp.float32),
        pltpu.SemaphoreType.DMA,
        pltpu.SemaphoreType.DMA,
    ],
)
def _emb_sc(x_hbm, w_hbm, out_hbm, idx_v, rows_v, gsem, osem):
    wid = lax.axis_index("s") * NC + lax.axis_index("c")
    p0 = wid * NPAIR

    pltpu.sync_copy(x_hbm.at[pl.ds(p0, NPAIR)], idx_v)

    def gather_start(p, rb):
        pltpu.async_copy(w_hbm.at[idx_v.at[p]], rows_v.at[rb], gsem)

    def gather_wait(p, rb):
        pltpu.make_async_copy(w_hbm.at[idx_v.at[p]], rows_v.at[rb], gsem).wait()

    def out_start(p, rb):
        pltpu.async_copy(rows_v.at[rb], out_hbm.at[p0 + p], osem)

    def out_wait(p, rb):
        pltpu.make_async_copy(rows_v.at[rb], out_hbm.at[p0 + p], osem).wait()

    def scale_buf(rb):
        rows = rows_v.at[rb]

        @plsc.parallel_loop(0, PAIR, unroll=4)
        def _(k):
            for i in range(D_MODEL // 16):
                sl = pl.ds(16 * i, 16)
                rows[k, sl] = rows[k, sl] * SCALE

    for rb in range(NRB):
        gather_start(rb, rb)

    def outer(g, _):
        for rb in range(NRB):
            p = g * NRB + rb
            gather_wait(p, rb)
            scale_buf(rb)
            out_start(p, rb)
            nxt = p + NRB

            @pl.when(nxt < NPAIR)
            def _():
                out_wait(p, rb)
                gather_start(nxt, rb)

        return 0

    lax.fori_loop(0, NPAIR // NRB, outer, 0)

    for rb in range(NRB):
        out_wait(NPAIR - NRB + rb, rb)


def kernel(x, weight):
    xp = jnp.concatenate([x, x[:, : HIST_PAD - HIST]], axis=1)
    xp = xp.reshape(BATCH // 2, PAIR)
    out = _emb_sc(xp, weight)
    return out.reshape(BATCH, HIST_PAD, D_MODEL)[:, :HIST, :]
